# baseline (speedup 1.0000x reference)
import jax
import jax.numpy as jnp
from jax.experimental import pallas as pl
from jax.experimental.pallas import tpu as pltpu

_NUM_CLASSES = 20
_SUBLANE = 8


def _round_up(a, m):
    return (a + m - 1) // m * m


def _head_body(x1_ref, x2_ref, w1_ref, w2_ref, b_ref, o_ref):
    acc = jnp.dot(x1_ref[...], w1_ref[...], preferred_element_type=jnp.float32)
    acc = acc + jnp.dot(x2_ref[...], w2_ref[...],
                        preferred_element_type=jnp.float32)
    acc = acc + b_ref[...]
    acc = jnp.maximum(acc, 0.0)
    o_ref[...] = acc[:, :_NUM_CLASSES]


def kernel(pooled_output, w_t_pad, b_pad):
    n, h = pooled_output.shape
    l_pad = w_t_pad.shape[1]
    hh = h // 2

    tm = min(4096, _round_up(n, _SUBLANE))
    n_pad = _round_up(n, tm)
    x = pooled_output
    if n_pad > n:
        x = jnp.pad(x, ((0, n_pad - n), (0, 0)))

    out = pl.pallas_call(
        _head_body,
        out_shape=jax.ShapeDtypeStruct((n_pad, _NUM_CLASSES), jnp.float32),
        grid=(n_pad // tm,),
        in_specs=[
            pl.BlockSpec((tm, hh), lambda i: (i, 0)),
            pl.BlockSpec((tm, hh), lambda i: (i, 1)),
            pl.BlockSpec((hh, l_pad), lambda i: (0, 0)),
            pl.BlockSpec((hh, l_pad), lambda i: (1, 0)),
            pl.BlockSpec((1, l_pad), lambda i: (0, 0)),
        ],
        out_specs=pl.BlockSpec((tm, _NUM_CLASSES), lambda i: (i, 0)),
        compiler_params=pltpu.CompilerParams(
            dimension_semantics=("parallel",),
        ),
    )(x, x, w_t_pad, w_t_pad, b_pad)

    return out[:n]
